# Initial kernel scaffold; baseline (speedup 1.0000x reference)
import jax
import jax.numpy as jnp
from jax.experimental import pallas as pl


def kernel(node_feat, query, edge_index, W_lin, b_lin, W_mlp1, b_mlp1, W_mlp2, b_mlp2, W_layer, b_layer):
    raise NotImplementedError("write your pallas kernel here")


# speedup vs baseline: 1.4915x; 1.4915x over previous
import functools
import jax
import jax.numpy as jnp
from jax import lax
from jax.experimental import pallas as pl

_N = 10000
_E = 320000
_D = 128
_FEAT = 256
_NUM_LAYER = 2
_K = 1000
_ES = 32000
_MAXB = 257

_BN = 2000


def _score_body(h, wlt, qb, w1, b1, w2, b2):
    heur = jnp.dot(h, wlt, preferred_element_type=jnp.float32) + qb
    x = h * heur
    hh = jax.nn.relu(jnp.dot(x, w1, preferred_element_type=jnp.float32) + b1)
    s = jnp.dot(hh, w2, preferred_element_type=jnp.float32) + b2
    return s


def _score_kernel(h_ref, wlt_ref, qb_ref, w1_ref, b1_ref, w2_ref, b2_ref,
                  score_ref, li_ref):
    h = h_ref[...]
    s = _score_body(h, wlt_ref[...], qb_ref[...], w1_ref[...], b1_ref[...],
                    w2_ref[...], b2_ref[...])
    score_ref[...] = s
    li_ref[...] = jax.nn.sigmoid(s) * h


def _update_kernel(h_ref, a0_ref, a1_ref, upd_ref, wl_ref, bl_ref,
                   wlt_ref, qb_ref, w1_ref, b1_ref, w2_ref, b2_ref,
                   hn_ref, score_ref, li_ref):
    agg = a0_ref[...] + a1_ref[...]
    ho = jax.nn.relu(jnp.dot(agg, wl_ref[...],
                             preferred_element_type=jnp.float32) + bl_ref[...])
    hn = h_ref[...] + upd_ref[...] * ho
    s = _score_body(hn, wlt_ref[...], qb_ref[...], w1_ref[...], b1_ref[...],
                    w2_ref[...], b2_ref[...])
    hn_ref[...] = hn
    score_ref[...] = s
    li_ref[...] = jax.nn.sigmoid(s) * hn


def _row_spec():
    return pl.BlockSpec((_BN, None), lambda i: (i, 0))


def _full_spec():
    return pl.BlockSpec(lambda i: (0, 0))


def _mk_score_call():
    grid = _N // _BN
    return pl.pallas_call(
        _score_kernel,
        grid=(grid,),
        in_specs=[
            pl.BlockSpec((_BN, _D), lambda i: (i, 0)),
            pl.BlockSpec((_D, _D), lambda i: (0, 0)),
            pl.BlockSpec((1, _D), lambda i: (0, 0)),
            pl.BlockSpec((_D, _FEAT), lambda i: (0, 0)),
            pl.BlockSpec((1, _FEAT), lambda i: (0, 0)),
            pl.BlockSpec((_FEAT, 1), lambda i: (0, 0)),
            pl.BlockSpec((1, 1), lambda i: (0, 0)),
        ],
        out_specs=[
            pl.BlockSpec((_BN, 1), lambda i: (i, 0)),
            pl.BlockSpec((_BN, _D), lambda i: (i, 0)),
        ],
        out_shape=[
            jax.ShapeDtypeStruct((_N, 1), jnp.float32),
            jax.ShapeDtypeStruct((_N, _D), jnp.float32),
        ],
    )


def _mk_update_call():
    grid = _N // _BN
    return pl.pallas_call(
        _update_kernel,
        grid=(grid,),
        in_specs=[
            pl.BlockSpec((_BN, _D), lambda i: (i, 0)),
            pl.BlockSpec((_BN, _D), lambda i: (i, 0)),
            pl.BlockSpec((_BN, _D), lambda i: (i, 0)),
            pl.BlockSpec((_BN, 1), lambda i: (i, 0)),
            pl.BlockSpec((_D, _D), lambda i: (0, 0)),
            pl.BlockSpec((1, _D), lambda i: (0, 0)),
            pl.BlockSpec((_D, _D), lambda i: (0, 0)),
            pl.BlockSpec((1, _D), lambda i: (0, 0)),
            pl.BlockSpec((_D, _FEAT), lambda i: (0, 0)),
            pl.BlockSpec((1, _FEAT), lambda i: (0, 0)),
            pl.BlockSpec((_FEAT, 1), lambda i: (0, 0)),
            pl.BlockSpec((1, 1), lambda i: (0, 0)),
        ],
        out_specs=[
            pl.BlockSpec((_BN, _D), lambda i: (i, 0)),
            pl.BlockSpec((_BN, 1), lambda i: (i, 0)),
            pl.BlockSpec((_BN, _D), lambda i: (i, 0)),
        ],
        out_shape=[
            jax.ShapeDtypeStruct((_N, _D), jnp.float32),
            jax.ShapeDtypeStruct((_N, 1), jnp.float32),
            jax.ShapeDtypeStruct((_N, _D), jnp.float32),
        ],
    )


def _select_thresholds(score, deg, src):
    top_vals, top_nodes = lax.top_k(score, _K)
    deg_top = deg[top_nodes]
    cum = jnp.cumsum(deg_top)
    p = jnp.searchsorted(cum, jnp.int32(_ES))
    is_full = p >= _K
    pc = jnp.minimum(p, _K - 1)
    b = top_nodes[pc]
    sb = top_vals[pc]
    csel = jnp.where(p > 0, cum[jnp.maximum(p - 1, 0)], 0)
    rem = jnp.clip(_ES - csel, 0, _MAXB - 1)
    bpos = jnp.nonzero(src == b, size=_MAXB, fill_value=_E)[0]
    et = jnp.where(is_full, _E, bpos[rem])
    return sb, b, et


def kernel(node_feat, query, edge_index, W_lin, b_lin, W_mlp1, b_mlp1,
           W_mlp2, b_mlp2, W_layer, b_layer):
    src = edge_index[0]
    dst = edge_index[1]

    wlt = W_lin[:_D]
    qb = (query @ W_lin[_D:] + b_lin)[None, :]
    b1 = b_mlp1[None, :]
    b2 = b_mlp2[None, :]

    score_call = _mk_score_call()
    update_call = _mk_update_call()

    deg = jax.ops.segment_sum(jnp.ones((_E,), jnp.int32), src,
                              num_segments=_N)
    upd = (deg > 0).astype(jnp.float32)[:, None]

    hidden = node_feat
    score2d, li = score_call(hidden, wlt, qb, W_mlp1, b1, W_mlp2, b2)
    score = score2d[:, 0]

    eids = jnp.arange(_E, dtype=jnp.int32)
    for i in range(_NUM_LAYER):
        sb, b, et = _select_thresholds(score, deg, src)
        mask = (score[src] > sb) | ((src == b) & (eids < et))
        msg = jnp.where(mask[:, None], li[src], 0.0)
        agg = jax.ops.segment_sum(msg, dst, num_segments=_N)
        zero = jnp.zeros_like(agg)
        hidden, score2d, li = update_call(
            hidden, agg, zero, upd, W_layer[i], b_layer[i][None, :],
            wlt, qb, W_mlp1, b1, W_mlp2, b2)
        score = score2d[:, 0]

    return score


# speedup vs baseline: 23.7537x; 15.9265x over previous
import functools
import jax
import jax.numpy as jnp
from jax import lax
from jax.experimental import pallas as pl
from jax.experimental.pallas import tpu as pltpu
from jax.experimental.pallas import tpu_sc as plsc

_N = 10000
_NP = 10240
_E = 320000
_D = 128
_FEAT = 256
_NUM_LAYER = 2
_K = 1000
_ES = 32000
_MAXB = 257

_BN = 2560
_NW = 32
_EPW = _E // _NW
_EC = 2000
_RPT = _NP // 16
_DUMMY = _N


def _score_body(h, wlt, qb, w1, b1, w2, b2):
    heur = jnp.dot(h, wlt, preferred_element_type=jnp.float32) + qb
    x = h * heur
    hh = jax.nn.relu(jnp.dot(x, w1, preferred_element_type=jnp.float32) + b1)
    return jnp.dot(hh, w2, preferred_element_type=jnp.float32) + b2


def _score_kernel(h_ref, wlt_ref, qb_ref, w1_ref, b1_ref, w2_ref, b2_ref,
                  score_ref, li_ref):
    h = h_ref[...]
    s = _score_body(h, wlt_ref[...], qb_ref[...], w1_ref[...], b1_ref[...],
                    w2_ref[...], b2_ref[...])
    score_ref[...] = s
    li_ref[...] = jax.nn.sigmoid(s) * h


def _update_kernel(h_ref, a0_ref, a1_ref, upd_ref, wl_ref, bl_ref,
                   wlt_ref, qb_ref, w1_ref, b1_ref, w2_ref, b2_ref,
                   hn_ref, score_ref, li_ref):
    agg = a0_ref[...] + a1_ref[...]
    ho = jax.nn.relu(jnp.dot(agg, wl_ref[...],
                             preferred_element_type=jnp.float32) + bl_ref[...])
    hn = h_ref[...] + upd_ref[...] * ho
    s = _score_body(hn, wlt_ref[...], qb_ref[...], w1_ref[...], b1_ref[...],
                    w2_ref[...], b2_ref[...])
    hn_ref[...] = hn
    score_ref[...] = s
    li_ref[...] = jax.nn.sigmoid(s) * hn


def _mk_score_call():
    grid = _NP // _BN
    return pl.pallas_call(
        _score_kernel,
        grid=(grid,),
        in_specs=[
            pl.BlockSpec((_BN, _D), lambda i: (i, 0)),
            pl.BlockSpec((_D, _D), lambda i: (0, 0)),
            pl.BlockSpec((1, _D), lambda i: (0, 0)),
            pl.BlockSpec((_D, _FEAT), lambda i: (0, 0)),
            pl.BlockSpec((1, _FEAT), lambda i: (0, 0)),
            pl.BlockSpec((_FEAT, 1), lambda i: (0, 0)),
            pl.BlockSpec((1, 1), lambda i: (0, 0)),
        ],
        out_specs=[
            pl.BlockSpec((_BN, 1), lambda i: (i, 0)),
            pl.BlockSpec((_BN, _D), lambda i: (i, 0)),
        ],
        out_shape=[
            jax.ShapeDtypeStruct((_NP, 1), jnp.float32),
            jax.ShapeDtypeStruct((_NP, _D), jnp.float32),
        ],
    )


def _mk_update_call():
    grid = _NP // _BN
    return pl.pallas_call(
        _update_kernel,
        grid=(grid,),
        in_specs=[
            pl.BlockSpec((_BN, _D), lambda i: (i, 0)),
            pl.BlockSpec((_BN, _D), lambda i: (i, 0)),
            pl.BlockSpec((_BN, _D), lambda i: (i, 0)),
            pl.BlockSpec((_BN, 1), lambda i: (i, 0)),
            pl.BlockSpec((_D, _D), lambda i: (0, 0)),
            pl.BlockSpec((1, _D), lambda i: (0, 0)),
            pl.BlockSpec((_D, _D), lambda i: (0, 0)),
            pl.BlockSpec((1, _D), lambda i: (0, 0)),
            pl.BlockSpec((_D, _FEAT), lambda i: (0, 0)),
            pl.BlockSpec((1, _FEAT), lambda i: (0, 0)),
            pl.BlockSpec((_FEAT, 1), lambda i: (0, 0)),
            pl.BlockSpec((1, 1), lambda i: (0, 0)),
        ],
        out_specs=[
            pl.BlockSpec((_BN, _D), lambda i: (i, 0)),
            pl.BlockSpec((_BN, 1), lambda i: (i, 0)),
            pl.BlockSpec((_BN, _D), lambda i: (i, 0)),
        ],
        out_shape=[
            jax.ShapeDtypeStruct((_NP, _D), jnp.float32),
            jax.ShapeDtypeStruct((_NP, 1), jnp.float32),
            jax.ShapeDtypeStruct((_NP, _D), jnp.float32),
        ],
    )


def _sc_edge_body(src_hbm, dst_hbm, score_hbm, sb_hbm, b_hbm, et_hbm,
                  li_hbm, zeros_hbm, agg_out,
                  src_v, dst_v, score_v, sb_v, b_v, et_v,
                  sel_s, sel_d, rows_v, agg_sh, sem):
    cid = lax.axis_index("c")
    sid = lax.axis_index("s")
    wid = cid * 16 + sid
    base = wid * _EPW

    pltpu.sync_copy(score_hbm, score_v)
    pltpu.sync_copy(sb_hbm, sb_v)
    pltpu.sync_copy(b_hbm, b_v)
    pltpu.sync_copy(et_hbm, et_v)
    pltpu.sync_copy(zeros_hbm.at[pl.ds(sid * _RPT, _RPT)],
                    agg_sh.at[pl.ds(sid * _RPT, _RPT)])

    sbv = sb_v[...]
    bv = b_v[...]
    etv = et_v[...]

    cnt = jnp.int32(0)
    for g in range(_EPW // _EC):
        gbase = base + g * _EC
        pltpu.sync_copy(src_hbm.at[pl.ds(gbase, _EC)], src_v)
        pltpu.sync_copy(dst_hbm.at[pl.ds(gbase, _EC)], dst_v)

        def body_a(i, off, gbase=gbase):
            s = src_v[pl.ds(i * 16, 16)]
            d = dst_v[pl.ds(i * 16, 16)]
            sc = plsc.load_gather(score_v, [s])
            eid = gbase + i * 16 + lax.iota(jnp.int32, 16)
            keep = (sc > sbv) | ((s == bv) & (eid < etv))
            ki = keep.astype(jnp.int32)
            pos = off + plsc.cumsum(ki) - 1
            plsc.store_scatter(sel_s, [pos], s, mask=keep)
            plsc.store_scatter(sel_d, [pos], d, mask=keep)
            return off + jnp.sum(ki)

        cnt = lax.fori_loop(0, _EC // 16, body_a, cnt)
    tail = cnt + lax.iota(jnp.int32, 16)
    plsc.store_scatter(sel_s, [tail], jnp.zeros((16,), jnp.int32))
    plsc.store_scatter(sel_d, [tail], jnp.full((16,), _DUMMY, jnp.int32))

    plsc.subcore_barrier()

    def body_b(j, carry):
        sv = sel_s[pl.ds(j * 16, 16)]
        dv = sel_d[pl.ds(j * 16, 16)]
        pltpu.async_copy(li_hbm.at[sv], rows_v, sem).wait()
        pltpu.sync_copy(rows_v, agg_sh.at[dv], add=True)
        return carry

    nch = (cnt + 15) // 16
    lax.fori_loop(0, nch, body_b, jnp.int32(0))

    plsc.subcore_barrier()
    pltpu.sync_copy(agg_sh.at[pl.ds(sid * _RPT, _RPT)],
                    agg_out.at[cid].at[pl.ds(sid * _RPT, _RPT)])


def _mk_sc_edge_call():
    mesh = plsc.VectorSubcoreMesh(core_axis_name="c", subcore_axis_name="s")
    return functools.partial(
        pl.kernel,
        out_type=jax.ShapeDtypeStruct((2, _NP, _D), jnp.float32),
        mesh=mesh,
        compiler_params=pltpu.CompilerParams(needs_layout_passes=False),
        scratch_types=[
            pltpu.VMEM((_EC,), jnp.int32),
            pltpu.VMEM((_EC,), jnp.int32),
            pltpu.VMEM((_NP,), jnp.float32),
            pltpu.VMEM((16,), jnp.float32),
            pltpu.VMEM((16,), jnp.int32),
            pltpu.VMEM((16,), jnp.int32),
            pltpu.VMEM((_EPW + 16,), jnp.int32),
            pltpu.VMEM((_EPW + 16,), jnp.int32),
            pltpu.VMEM((16, _D), jnp.float32),
            pltpu.VMEM_SHARED((_NP, _D), jnp.float32),
            pltpu.SemaphoreType.DMA,
        ],
    )(_sc_edge_body)


def _select_thresholds(score, deg, src):
    top_vals, top_nodes = lax.top_k(score, _K)
    deg_top = deg[top_nodes]
    cum = jnp.cumsum(deg_top)
    p = jnp.searchsorted(cum, jnp.int32(_ES))
    is_full = p >= _K
    pc = jnp.minimum(p, _K - 1)
    b = top_nodes[pc]
    sb = top_vals[pc]
    csel = jnp.where(p > 0, cum[jnp.maximum(p - 1, 0)], 0)
    rem = jnp.clip(_ES - csel, 0, _MAXB - 1)
    bpos = jnp.nonzero(src == b, size=_MAXB, fill_value=_E)[0]
    et = jnp.where(is_full, _E, bpos[rem])
    return sb, b, et


def kernel(node_feat, query, edge_index, W_lin, b_lin, W_mlp1, b_mlp1,
           W_mlp2, b_mlp2, W_layer, b_layer):
    src = edge_index[0]
    dst = edge_index[1]

    wlt = W_lin[:_D]
    qb = (query @ W_lin[_D:] + b_lin)[None, :]
    b1 = b_mlp1[None, :]
    b2 = b_mlp2[None, :]

    score_call = _mk_score_call()
    update_call = _mk_update_call()
    sc_edge = _mk_sc_edge_call()

    hid0 = jnp.pad(node_feat, ((0, _NP - _N), (0, 0)))
    zeros = jnp.zeros((_NP, _D), jnp.float32)

    deg = jax.ops.segment_sum(jnp.ones((_E,), jnp.int32), src,
                              num_segments=_NP)
    upd = (deg > 0).astype(jnp.float32)[:, None]

    hidden = hid0
    score2d, li = score_call(hidden, wlt, qb, W_mlp1, b1, W_mlp2, b2)

    for i in range(_NUM_LAYER):
        score = score2d[:_N, 0]
        sb, b, et = _select_thresholds(score, deg, src)
        agg = sc_edge(src, dst, score2d[:, 0],
                      jnp.full((16,), sb, jnp.float32),
                      jnp.full((16,), b, jnp.int32),
                      jnp.full((16,), et, jnp.int32),
                      li, zeros)
        hidden, score2d, li = update_call(
            hidden, agg[0], agg[1], upd, W_layer[i], b_layer[i][None, :],
            wlt, qb, W_mlp1, b1, W_mlp2, b2)

    return score2d[:_N, 0]


# speedup vs baseline: 24.9677x; 1.0511x over previous
import functools
import jax
import jax.numpy as jnp
from jax import lax
from jax.experimental import pallas as pl
from jax.experimental.pallas import tpu as pltpu
from jax.experimental.pallas import tpu_sc as plsc

_N = 10000
_NP = 10240
_E = 320000
_D = 128
_FEAT = 256
_NUM_LAYER = 2
_K = 1000
_ES = 32000
_MAXB = 257

_BN = 2560
_NW = 32
_EPW = _E // _NW
_EC = 2000
_RPT = _NP // 16
_DUMMY = _N


def _score_body(h, wlt, qb, w1, b1, w2, b2):
    heur = jnp.dot(h, wlt, preferred_element_type=jnp.float32) + qb
    x = h * heur
    hh = jax.nn.relu(jnp.dot(x, w1, preferred_element_type=jnp.float32) + b1)
    return jnp.dot(hh, w2, preferred_element_type=jnp.float32) + b2


def _score_kernel(h_ref, wlt_ref, qb_ref, w1_ref, b1_ref, w2_ref, b2_ref,
                  score_ref, li_ref):
    h = h_ref[...]
    s = _score_body(h, wlt_ref[...], qb_ref[...], w1_ref[...], b1_ref[...],
                    w2_ref[...], b2_ref[...])
    score_ref[...] = s
    li_ref[...] = jax.nn.sigmoid(s) * h


def _update_kernel(h_ref, a0_ref, a1_ref, upd_ref, wl_ref, bl_ref,
                   wlt_ref, qb_ref, w1_ref, b1_ref, w2_ref, b2_ref,
                   hn_ref, score_ref, li_ref):
    agg = a0_ref[...] + a1_ref[...]
    ho = jax.nn.relu(jnp.dot(agg, wl_ref[...],
                             preferred_element_type=jnp.float32) + bl_ref[...])
    hn = h_ref[...] + upd_ref[...] * ho
    s = _score_body(hn, wlt_ref[...], qb_ref[...], w1_ref[...], b1_ref[...],
                    w2_ref[...], b2_ref[...])
    hn_ref[...] = hn
    score_ref[...] = s
    li_ref[...] = jax.nn.sigmoid(s) * hn


def _mk_score_call():
    grid = _NP // _BN
    return pl.pallas_call(
        _score_kernel,
        grid=(grid,),
        in_specs=[
            pl.BlockSpec((_BN, _D), lambda i: (i, 0)),
            pl.BlockSpec((_D, _D), lambda i: (0, 0)),
            pl.BlockSpec((1, _D), lambda i: (0, 0)),
            pl.BlockSpec((_D, _FEAT), lambda i: (0, 0)),
            pl.BlockSpec((1, _FEAT), lambda i: (0, 0)),
            pl.BlockSpec((_FEAT, 1), lambda i: (0, 0)),
            pl.BlockSpec((1, 1), lambda i: (0, 0)),
        ],
        out_specs=[
            pl.BlockSpec((_BN, 1), lambda i: (i, 0)),
            pl.BlockSpec((_BN, _D), lambda i: (i, 0)),
        ],
        out_shape=[
            jax.ShapeDtypeStruct((_NP, 1), jnp.float32),
            jax.ShapeDtypeStruct((_NP, _D), jnp.float32),
        ],
    )


def _mk_update_call():
    grid = _NP // _BN
    return pl.pallas_call(
        _update_kernel,
        grid=(grid,),
        in_specs=[
            pl.BlockSpec((_BN, _D), lambda i: (i, 0)),
            pl.BlockSpec((_BN, _D), lambda i: (i, 0)),
            pl.BlockSpec((_BN, _D), lambda i: (i, 0)),
            pl.BlockSpec((_BN, 1), lambda i: (i, 0)),
            pl.BlockSpec((_D, _D), lambda i: (0, 0)),
            pl.BlockSpec((1, _D), lambda i: (0, 0)),
            pl.BlockSpec((_D, _D), lambda i: (0, 0)),
            pl.BlockSpec((1, _D), lambda i: (0, 0)),
            pl.BlockSpec((_D, _FEAT), lambda i: (0, 0)),
            pl.BlockSpec((1, _FEAT), lambda i: (0, 0)),
            pl.BlockSpec((_FEAT, 1), lambda i: (0, 0)),
            pl.BlockSpec((1, 1), lambda i: (0, 0)),
        ],
        out_specs=[
            pl.BlockSpec((_BN, _D), lambda i: (i, 0)),
            pl.BlockSpec((_BN, 1), lambda i: (i, 0)),
            pl.BlockSpec((_BN, _D), lambda i: (i, 0)),
        ],
        out_shape=[
            jax.ShapeDtypeStruct((_NP, _D), jnp.float32),
            jax.ShapeDtypeStruct((_NP, 1), jnp.float32),
            jax.ShapeDtypeStruct((_NP, _D), jnp.float32),
        ],
    )


def _sc_edge_body(src_hbm, dst_hbm, score_hbm, sb_hbm, b_hbm, et_hbm,
                  li_hbm, zeros_hbm, agg_out,
                  src_v, dst_v, score_v, sb_v, b_v, et_v,
                  sel_s, sel_d, rows_v, agg_sh, sem):
    cid = lax.axis_index("c")
    sid = lax.axis_index("s")
    wid = cid * 16 + sid
    base = wid * _EPW

    pltpu.sync_copy(score_hbm, score_v)
    pltpu.sync_copy(sb_hbm, sb_v)
    pltpu.sync_copy(b_hbm, b_v)
    pltpu.sync_copy(et_hbm, et_v)
    pltpu.sync_copy(zeros_hbm.at[pl.ds(sid * _RPT, _RPT)],
                    agg_sh.at[pl.ds(sid * _RPT, _RPT)])

    sbv = sb_v[...]
    bv = b_v[...]
    etv = et_v[...]

    cnt = jnp.int32(0)
    for g in range(_EPW // _EC):
        gbase = base + g * _EC
        pltpu.sync_copy(src_hbm.at[pl.ds(gbase, _EC)], src_v)
        pltpu.sync_copy(dst_hbm.at[pl.ds(gbase, _EC)], dst_v)

        def body_a(i, off, gbase=gbase):
            s = src_v[pl.ds(i * 16, 16)]
            d = dst_v[pl.ds(i * 16, 16)]
            sc = plsc.load_gather(score_v, [s])
            eid = gbase + i * 16 + lax.iota(jnp.int32, 16)
            keep = (sc > sbv) | ((s == bv) & (eid < etv))
            ki = keep.astype(jnp.int32)
            pos = off + plsc.cumsum(ki) - 1
            plsc.store_scatter(sel_s, [pos], s, mask=keep)
            plsc.store_scatter(sel_d, [pos], d, mask=keep)
            return off + jnp.sum(ki)

        cnt = lax.fori_loop(0, _EC // 16, body_a, cnt)
    for q in range(4):
        tail = cnt + q * 16 + lax.iota(jnp.int32, 16)
        plsc.store_scatter(sel_s, [tail], jnp.zeros((16,), jnp.int32))
        plsc.store_scatter(sel_d, [tail], jnp.full((16,), _DUMMY, jnp.int32))

    plsc.subcore_barrier()

    def body_b(j, carry):
        pltpu.async_copy(li_hbm.at[sel_s.at[pl.ds(j * 64, 64)]],
                         rows_v, sem).wait()
        for q in range(4):
            dv = sel_d[pl.ds(j * 64 + q * 16, 16)]
            pltpu.sync_copy(rows_v.at[pl.ds(q * 16, 16)],
                            agg_sh.at[dv], add=True)
        return carry

    nch = (cnt + 63) // 64
    lax.fori_loop(0, nch, body_b, jnp.int32(0))

    plsc.subcore_barrier()
    pltpu.sync_copy(agg_sh.at[pl.ds(sid * _RPT, _RPT)],
                    agg_out.at[cid].at[pl.ds(sid * _RPT, _RPT)])


def _mk_sc_edge_call():
    mesh = plsc.VectorSubcoreMesh(core_axis_name="c", subcore_axis_name="s")
    return functools.partial(
        pl.kernel,
        out_type=jax.ShapeDtypeStruct((2, _NP, _D), jnp.float32),
        mesh=mesh,
        compiler_params=pltpu.CompilerParams(needs_layout_passes=False),
        scratch_types=[
            pltpu.VMEM((_EC,), jnp.int32),
            pltpu.VMEM((_EC,), jnp.int32),
            pltpu.VMEM((_NP,), jnp.float32),
            pltpu.VMEM((16,), jnp.float32),
            pltpu.VMEM((16,), jnp.int32),
            pltpu.VMEM((16,), jnp.int32),
            pltpu.VMEM((_EPW + 64,), jnp.int32),
            pltpu.VMEM((_EPW + 64,), jnp.int32),
            pltpu.VMEM((64, _D), jnp.float32),
            pltpu.VMEM_SHARED((_NP, _D), jnp.float32),
            pltpu.SemaphoreType.DMA,
        ],
    )(_sc_edge_body)


def _select_thresholds(score, deg, src):
    top_vals, top_nodes = lax.top_k(score, _K)
    deg_top = deg[top_nodes]
    cum = jnp.cumsum(deg_top)
    p = jnp.searchsorted(cum, jnp.int32(_ES))
    is_full = p >= _K
    pc = jnp.minimum(p, _K - 1)
    b = top_nodes[pc]
    sb = top_vals[pc]
    csel = jnp.where(p > 0, cum[jnp.maximum(p - 1, 0)], 0)
    rem = jnp.maximum(_ES - csel, 0)
    bcnt = jnp.cumsum((src == b).astype(jnp.int32))
    et = jnp.where(is_full, _E,
                   jnp.searchsorted(bcnt, rem, side="right"))
    return sb, b, et


def kernel(node_feat, query, edge_index, W_lin, b_lin, W_mlp1, b_mlp1,
           W_mlp2, b_mlp2, W_layer, b_layer):
    src = edge_index[0]
    dst = edge_index[1]

    wlt = W_lin[:_D]
    qb = (query @ W_lin[_D:] + b_lin)[None, :]
    b1 = b_mlp1[None, :]
    b2 = b_mlp2[None, :]

    score_call = _mk_score_call()
    update_call = _mk_update_call()
    sc_edge = _mk_sc_edge_call()

    hid0 = jnp.pad(node_feat, ((0, _NP - _N), (0, 0)))
    zeros = jnp.zeros((_NP, _D), jnp.float32)

    deg = jax.ops.segment_sum(jnp.ones((_E,), jnp.int32), src,
                              num_segments=_NP)
    upd = (deg > 0).astype(jnp.float32)[:, None]

    hidden = hid0
    score2d, li = score_call(hidden, wlt, qb, W_mlp1, b1, W_mlp2, b2)

    for i in range(_NUM_LAYER):
        score = score2d[:_N, 0]
        sb, b, et = _select_thresholds(score, deg, src)
        agg = sc_edge(src, dst, score2d[:, 0],
                      jnp.full((16,), sb, jnp.float32),
                      jnp.full((16,), b, jnp.int32),
                      jnp.full((16,), et, jnp.int32),
                      li, zeros)
        hidden, score2d, li = update_call(
            hidden, agg[0], agg[1], upd, W_layer[i], b_layer[i][None, :],
            wlt, qb, W_mlp1, b1, W_mlp2, b2)

    return score2d[:_N, 0]


# speedup vs baseline: 25.9160x; 1.0380x over previous
import functools
import jax
import jax.numpy as jnp
from jax import lax
from jax.experimental import pallas as pl
from jax.experimental.pallas import tpu as pltpu
from jax.experimental.pallas import tpu_sc as plsc

_N = 10000
_NP = 10240
_E = 320000
_D = 128
_FEAT = 256
_NUM_LAYER = 2
_K = 1000
_ES = 32000
_MAXB = 257

_BN = 2560
_NW = 32
_EPW = _E // _NW
_EC = 2000
_SELCAP = 8192 + 64
_RPT = _NP // 16
_DUMMY = _N


def _score_body(h, wlt, qb, w1, b1, w2, b2):
    heur = jnp.dot(h, wlt, preferred_element_type=jnp.float32) + qb
    x = h * heur
    hh = jax.nn.relu(jnp.dot(x, w1, preferred_element_type=jnp.float32) + b1)
    return jnp.dot(hh, w2, preferred_element_type=jnp.float32) + b2


def _score_kernel(h_ref, wlt_ref, qb_ref, w1_ref, b1_ref, w2_ref, b2_ref,
                  score_ref, li_ref):
    h = h_ref[...]
    s = _score_body(h, wlt_ref[...], qb_ref[...], w1_ref[...], b1_ref[...],
                    w2_ref[...], b2_ref[...])
    score_ref[...] = s
    li_ref[...] = jax.nn.sigmoid(s) * h


def _update_kernel(h_ref, a0_ref, a1_ref, upd_ref, wl_ref, bl_ref,
                   wlt_ref, qb_ref, w1_ref, b1_ref, w2_ref, b2_ref,
                   hn_ref, score_ref, li_ref):
    agg = a0_ref[...] + a1_ref[...]
    ho = jax.nn.relu(jnp.dot(agg, wl_ref[...],
                             preferred_element_type=jnp.float32) + bl_ref[...])
    hn = h_ref[...] + upd_ref[...] * ho
    s = _score_body(hn, wlt_ref[...], qb_ref[...], w1_ref[...], b1_ref[...],
                    w2_ref[...], b2_ref[...])
    hn_ref[...] = hn
    score_ref[...] = s
    li_ref[...] = jax.nn.sigmoid(s) * hn


def _mk_score_call():
    grid = _NP // _BN
    return pl.pallas_call(
        _score_kernel,
        grid=(grid,),
        in_specs=[
            pl.BlockSpec((_BN, _D), lambda i: (i, 0)),
            pl.BlockSpec((_D, _D), lambda i: (0, 0)),
            pl.BlockSpec((1, _D), lambda i: (0, 0)),
            pl.BlockSpec((_D, _FEAT), lambda i: (0, 0)),
            pl.BlockSpec((1, _FEAT), lambda i: (0, 0)),
            pl.BlockSpec((_FEAT, 1), lambda i: (0, 0)),
            pl.BlockSpec((1, 1), lambda i: (0, 0)),
        ],
        out_specs=[
            pl.BlockSpec((_BN, 1), lambda i: (i, 0)),
            pl.BlockSpec((_BN, _D), lambda i: (i, 0)),
        ],
        out_shape=[
            jax.ShapeDtypeStruct((_NP, 1), jnp.float32),
            jax.ShapeDtypeStruct((_NP, _D), jnp.float32),
        ],
    )


def _mk_update_call():
    grid = _NP // _BN
    return pl.pallas_call(
        _update_kernel,
        grid=(grid,),
        in_specs=[
            pl.BlockSpec((_BN, _D), lambda i: (i, 0)),
            pl.BlockSpec((_BN, _D), lambda i: (i, 0)),
            pl.BlockSpec((_BN, _D), lambda i: (i, 0)),
            pl.BlockSpec((_BN, 1), lambda i: (i, 0)),
            pl.BlockSpec((_D, _D), lambda i: (0, 0)),
            pl.BlockSpec((1, _D), lambda i: (0, 0)),
            pl.BlockSpec((_D, _D), lambda i: (0, 0)),
            pl.BlockSpec((1, _D), lambda i: (0, 0)),
            pl.BlockSpec((_D, _FEAT), lambda i: (0, 0)),
            pl.BlockSpec((1, _FEAT), lambda i: (0, 0)),
            pl.BlockSpec((_FEAT, 1), lambda i: (0, 0)),
            pl.BlockSpec((1, 1), lambda i: (0, 0)),
        ],
        out_specs=[
            pl.BlockSpec((_BN, _D), lambda i: (i, 0)),
            pl.BlockSpec((_BN, 1), lambda i: (i, 0)),
            pl.BlockSpec((_BN, _D), lambda i: (i, 0)),
        ],
        out_shape=[
            jax.ShapeDtypeStruct((_NP, _D), jnp.float32),
            jax.ShapeDtypeStruct((_NP, 1), jnp.float32),
            jax.ShapeDtypeStruct((_NP, _D), jnp.float32),
        ],
    )


def _sc_edge_body(src_hbm, dst_hbm, score_hbm, sb_hbm, b_hbm, et_hbm,
                  li_hbm, zeros_hbm, agg_out,
                  src_v, dst_v, score_v, sb_v, b_v, et_v,
                  sel_s, sel_d, rows_v, rows_w, agg_sh, sem, sem2):
    cid = lax.axis_index("c")
    sid = lax.axis_index("s")
    wid = cid * 16 + sid
    base = wid * _EPW

    pltpu.sync_copy(score_hbm, score_v)
    pltpu.sync_copy(sb_hbm, sb_v)
    pltpu.sync_copy(b_hbm, b_v)
    pltpu.sync_copy(et_hbm, et_v)
    pltpu.sync_copy(zeros_hbm.at[pl.ds(sid * _RPT, _RPT)],
                    agg_sh.at[pl.ds(sid * _RPT, _RPT)])

    sbv = sb_v[...]
    bv = b_v[...]
    etv = et_v[...]

    offv = jnp.zeros((16,), jnp.int32)
    for g in range(_EPW // _EC):
        gbase = base + g * _EC
        pltpu.sync_copy(src_hbm.at[pl.ds(gbase, _EC)], src_v)
        pltpu.sync_copy(dst_hbm.at[pl.ds(gbase, _EC)], dst_v)

        def body_a(i, off, gbase=gbase):
            s = src_v[pl.ds(i * 16, 16)]
            d = dst_v[pl.ds(i * 16, 16)]
            sc = plsc.load_gather(score_v, [s])
            eid = gbase + i * 16 + lax.iota(jnp.int32, 16)
            keep = (sc > sbv) | ((s == bv) & (eid < etv))
            ki = keep.astype(jnp.int32)
            pos = off + plsc.cumsum(ki) - 1
            plsc.store_scatter(sel_s, [pos], s, mask=keep)
            plsc.store_scatter(sel_d, [pos], d, mask=keep)
            return off + jnp.sum(ki)

        offv = lax.fori_loop(0, _EC // 16, body_a, offv)
    cnt = jnp.max(offv)
    for q in range(4):
        tail = cnt + q * 16 + lax.iota(jnp.int32, 16)
        plsc.store_scatter(sel_s, [tail], jnp.zeros((16,), jnp.int32))
        plsc.store_scatter(sel_d, [tail], jnp.full((16,), _DUMMY, jnp.int32))

    plsc.subcore_barrier()

    nch = (cnt + 63) // 64

    def _issue(buf, sm, batch):
        pltpu.async_copy(li_hbm.at[sel_s.at[pl.ds(batch * 64, 64)]], buf, sm)

    def _drain(buf, sm):
        pltpu.make_async_copy(li_hbm.at[pl.ds(0, 64)], buf, sm).wait()

    def _scatter(buf, batch):
        for q in range(4):
            dv = sel_d[pl.ds(batch * 64 + q * 16, 16)]
            pltpu.sync_copy(buf.at[pl.ds(q * 16, 16)],
                            agg_sh.at[dv], add=True)

    @pl.when(nch > 0)
    def _():
        _issue(rows_v, sem, 0)

    @pl.when(nch > 1)
    def _():
        _issue(rows_w, sem2, 1)

    def body_b(j, carry):
        b0 = 2 * j
        _drain(rows_v, sem)
        _scatter(rows_v, b0)

        @pl.when(b0 + 2 < nch)
        def _():
            _issue(rows_v, sem, b0 + 2)

        @pl.when(b0 + 1 < nch)
        def _():
            _drain(rows_w, sem2)
            _scatter(rows_w, b0 + 1)

            @pl.when(b0 + 3 < nch)
            def _():
                _issue(rows_w, sem2, b0 + 3)

        return carry

    lax.fori_loop(0, (nch + 1) // 2, body_b, jnp.int32(0))

    plsc.subcore_barrier()
    pltpu.sync_copy(agg_sh.at[pl.ds(sid * _RPT, _RPT)],
                    agg_out.at[cid].at[pl.ds(sid * _RPT, _RPT)])


def _mk_sc_edge_call():
    mesh = plsc.VectorSubcoreMesh(core_axis_name="c", subcore_axis_name="s")
    return functools.partial(
        pl.kernel,
        out_type=jax.ShapeDtypeStruct((2, _NP, _D), jnp.float32),
        mesh=mesh,
        compiler_params=pltpu.CompilerParams(needs_layout_passes=False),
        scratch_types=[
            pltpu.VMEM((_EC,), jnp.int32),
            pltpu.VMEM((_EC,), jnp.int32),
            pltpu.VMEM((_NP,), jnp.float32),
            pltpu.VMEM((16,), jnp.float32),
            pltpu.VMEM((16,), jnp.int32),
            pltpu.VMEM((16,), jnp.int32),
            pltpu.VMEM((_SELCAP,), jnp.int32),
            pltpu.VMEM((_SELCAP,), jnp.int32),
            pltpu.VMEM((64, _D), jnp.float32),
            pltpu.VMEM((64, _D), jnp.float32),
            pltpu.VMEM_SHARED((_NP, _D), jnp.float32),
            pltpu.SemaphoreType.DMA,
            pltpu.SemaphoreType.DMA,
        ],
    )(_sc_edge_body)


def _select_thresholds(score, deg, src):
    top_vals, top_nodes = lax.top_k(score, _K)
    deg_top = deg[top_nodes]
    cum = jnp.cumsum(deg_top)
    p = jnp.searchsorted(cum, jnp.int32(_ES))
    is_full = p >= _K
    pc = jnp.minimum(p, _K - 1)
    b = top_nodes[pc]
    sb = top_vals[pc]
    csel = jnp.where(p > 0, cum[jnp.maximum(p - 1, 0)], 0)
    rem = jnp.maximum(_ES - csel, 0)
    bcnt = jnp.cumsum((src == b).astype(jnp.int32))
    et = jnp.where(is_full, _E,
                   jnp.searchsorted(bcnt, rem, side="right"))
    return sb, b, et


def kernel(node_feat, query, edge_index, W_lin, b_lin, W_mlp1, b_mlp1,
           W_mlp2, b_mlp2, W_layer, b_layer):
    src = edge_index[0]
    dst = edge_index[1]

    wlt = W_lin[:_D]
    qb = (query @ W_lin[_D:] + b_lin)[None, :]
    b1 = b_mlp1[None, :]
    b2 = b_mlp2[None, :]

    score_call = _mk_score_call()
    update_call = _mk_update_call()
    sc_edge = _mk_sc_edge_call()

    hid0 = jnp.pad(node_feat, ((0, _NP - _N), (0, 0)))
    zeros = jnp.zeros((_NP, _D), jnp.float32)

    deg = jax.ops.segment_sum(jnp.ones((_E,), jnp.int32), src,
                              num_segments=_NP)
    upd = (deg > 0).astype(jnp.float32)[:, None]

    hidden = hid0
    score2d, li = score_call(hidden, wlt, qb, W_mlp1, b1, W_mlp2, b2)

    for i in range(_NUM_LAYER):
        score = score2d[:_N, 0]
        sb, b, et = _select_thresholds(score, deg, src)
        agg = sc_edge(src, dst, score2d[:, 0],
                      jnp.full((16,), sb, jnp.float32),
                      jnp.full((16,), b, jnp.int32),
                      jnp.full((16,), et, jnp.int32),
                      li, zeros)
        hidden, score2d, li = update_call(
            hidden, agg[0], agg[1], upd, W_layer[i], b_layer[i][None, :],
            wlt, qb, W_mlp1, b1, W_mlp2, b2)

    return score2d[:_N, 0]
